# baseline (speedup 1.0000x reference)
import functools

import jax
import jax.numpy as jnp
import numpy as np
from jax import lax
from jax.experimental import pallas as pl
from jax.experimental.pallas import tpu as pltpu
from jax.experimental.pallas import tpu_sc as plsc

_P = 0.5


def _plan_indices(seq_len: int) -> np.ndarray:
    rng = np.random.default_rng(0)
    pa = rng.uniform(0.0, 1.0, size=6)
    idx_list = []
    L = seq_len
    if pa[0] < _P:
        start = int(rng.uniform(0.0, L * 0.1))
        idx = np.arange(start, L, dtype=np.int64)
        idx_list.append(idx)
        L = idx.shape[0]
    if pa[1] < _P:
        end = int(rng.uniform(0.0, L * 0.1))
        idx = np.arange(0, L - end, dtype=np.int64)
        idx_list.append(idx)
        L = idx.shape[0]
    if pa[2] < _P:
        delta = float(np.float16(rng.uniform(0.8, 1.2)))
        idx = np.floor(np.arange(0.0, L, delta)).astype(np.int64)
        idx = np.clip(idx, 0, L - 1)
        idx_list.append(idx)
        L = idx.shape[0]
    if pa[3] < _P:
        margin = int(0.1 * L)
        center = int(rng.uniform(margin, L - margin))
        delta = float(np.float16(rng.uniform(0.5, 1.5)))
        mid = np.arange(center - margin, center + margin, delta).astype(np.int64)
        mid = np.clip(mid, 0, L - 1)
        idx = np.concatenate([
            np.arange(0, center - margin, dtype=np.int64),
            mid,
            np.arange(center + margin, L, dtype=np.int64),
        ])
        idx_list.append(idx)
        L = idx.shape[0]
    if pa[4] < _P:
        margin = int(0.1 * L)
        center = int(rng.uniform(margin, L - margin))
        mid = rng.permutation(np.arange(center - margin, center + margin, dtype=np.int64))
        idx = np.concatenate([
            np.arange(0, center - margin, dtype=np.int64),
            mid,
            np.arange(center + margin, L, dtype=np.int64),
        ])
        idx_list.append(idx)
        L = idx.shape[0]
    if pa[4] < _P:
        margin = int(0.1 * L)
        center = int(rng.uniform(margin, L - margin))
        idx = np.concatenate([
            np.arange(0, center - margin, dtype=np.int64),
            np.arange(center - margin, center + margin, dtype=np.int64)[::-1],
            np.arange(center + margin, L, dtype=np.int64),
        ])
        idx_list.append(idx)
        L = idx.shape[0]
    final = idx_list[0]
    for idx in idx_list[1:]:
        final = final[idx]
    return final.astype(np.int32)


_SEQ_LEN = 4096
_FEAT = 512
_IDX_NP = _plan_indices(_SEQ_LEN)
_OUT_LEN = int(_IDX_NP.shape[0])

_NUM_WORKERS = 32
_ROWS_PER_WORKER = -(-_OUT_LEN // (8 * _NUM_WORKERS)) * 8
assert _ROWS_PER_WORKER <= 128
_LAST_BASE = _OUT_LEN - _ROWS_PER_WORKER
assert _LAST_BASE % 8 == 0 and _OUT_LEN % 8 == 0

_CHUNK_OFFS = (0, 16, 48, _ROWS_PER_WORKER)
_N_CHUNKS = len(_CHUNK_OFFS) - 1
assert all(o % 8 == 0 for o in _CHUNK_OFFS)

_LANES = 16
_GEN_LEN = -(-_ROWS_PER_WORKER // _LANES) * _LANES


def _formula_params():
    rng = np.random.default_rng(0)
    pa = rng.uniform(0.0, 1.0, size=6)
    assert pa[0] >= _P and pa[1] < _P and pa[2] < _P and pa[3] < _P and pa[4] >= _P
    L = _SEQ_LEN
    end = int(rng.uniform(0.0, L * 0.1))
    L1 = L - end
    delta1 = float(np.float16(rng.uniform(0.8, 1.2)))
    L2 = len(np.arange(0.0, L1, delta1))
    margin = int(0.1 * L2)
    center = int(rng.uniform(margin, L2 - margin))
    delta3 = float(np.float16(rng.uniform(0.5, 1.5)))
    midlen = len(np.arange(center - margin, center + margin, delta3))
    p1 = center - margin
    shift = 2 * margin - midlen
    return dict(L1=L1, L2=L2, delta1=delta1, delta3=delta3,
                p1=p1, midlen=midlen, shift=shift)


_FP = _formula_params()


def _simulate_f32(j):
    f32 = np.float32
    jf = j.astype(f32)
    mid = (jf - f32(_FP["p1"])) * f32(_FP["delta3"]) + f32(_FP["p1"])
    v_mid = np.minimum(mid.astype(np.int32), _FP["L2"] - 1)
    v = np.where(j < _FP["p1"], j,
                 np.where(j < _FP["p1"] + _FP["midlen"], v_mid,
                          j + _FP["shift"]))
    return np.minimum((v.astype(f32) * f32(_FP["delta1"])).astype(np.int32),
                      _FP["L1"] - 1)


_ALL_J = np.arange(0, _OUT_LEN + (_GEN_LEN - _ROWS_PER_WORKER), dtype=np.int32)
_SIM = _simulate_f32(_ALL_J)
assert np.array_equal(_SIM[:_OUT_LEN], _IDX_NP)
assert _SIM.min() >= 0 and _SIM.max() < _SEQ_LEN


def _gen_indices(idx_v, base):
    f32, s32 = jnp.float32, jnp.int32
    for t in range(_GEN_LEN // _LANES):
        j = lax.iota(s32, _LANES) + (base + t * _LANES)
        jf = j.astype(f32)
        mid = (jf - f32(_FP["p1"])) * f32(_FP["delta3"]) + f32(_FP["p1"])
        v_mid = jnp.minimum(mid.astype(s32), _FP["L2"] - 1)
        v = jnp.where(j < _FP["p1"], j,
                      jnp.where(j < _FP["p1"] + _FP["midlen"], v_mid,
                                j + _FP["shift"]))
        fin = jnp.minimum((v.astype(f32) * f32(_FP["delta1"])).astype(s32),
                          _FP["L1"] - 1)
        idx_v[pl.ds(t * _LANES, _LANES)] = fin


def _sc_gather(a_hbm, b_hbm, out_hbm, idx_v, rows_a, rows_b, gsems, wsems):
    info = plsc.get_sparse_core_info()
    wid = lax.axis_index("s") * info.num_cores + lax.axis_index("c")
    base = jnp.where(wid == _NUM_WORKERS - 1, _LAST_BASE,
                     wid * _ROWS_PER_WORKER)
    _gen_indices(idx_v, base)
    gathers = []
    for c in range(_N_CHUNKS):
        off, n = _CHUNK_OFFS[c], _CHUNK_OFFS[c + 1] - _CHUNK_OFFS[c]
        sl = pl.ds(off, n)
        for j, (src, buf) in enumerate(((a_hbm, rows_a), (b_hbm, rows_b))):
            cp = pltpu.async_copy(src.at[idx_v.at[sl]], buf.at[sl],
                                  gsems.at[c * 2 + j])
            gathers.append((cp, buf, off, n, j))
    writes = []
    for k, (cp, buf, off, n, j) in enumerate(gathers):
        cp.wait()
        writes.append(pltpu.async_copy(buf.at[pl.ds(off, n)],
                                       out_hbm.at[pl.ds(base + off + j * _OUT_LEN, n)],
                                       wsems.at[k]))
    for w in writes:
        w.wait()


@jax.jit
def kernel(seq_a, seq_b):
    mesh = plsc.VectorSubcoreMesh(core_axis_name="c", subcore_axis_name="s")
    out = pl.kernel(
        _sc_gather,
        mesh=mesh,
        out_type=jax.ShapeDtypeStruct((2 * _OUT_LEN, _FEAT), jnp.float32),
        scratch_types=[
            pltpu.VMEM((_GEN_LEN,), jnp.int32),
            pltpu.VMEM((_ROWS_PER_WORKER, _FEAT), jnp.float32),
            pltpu.VMEM((_ROWS_PER_WORKER, _FEAT), jnp.float32),
            pltpu.SemaphoreType.DMA((_N_CHUNKS * 2,)),
            pltpu.SemaphoreType.DMA((_N_CHUNKS * 2,)),
        ],
    )(seq_a, seq_b)
    return out.reshape(2, _OUT_LEN, _FEAT)


# speedup vs baseline: 1.0137x; 1.0137x over previous
import functools

import jax
import jax.numpy as jnp
import numpy as np
from jax import lax
from jax.experimental import pallas as pl
from jax.experimental.pallas import tpu as pltpu
from jax.experimental.pallas import tpu_sc as plsc

_P = 0.5


def _plan_indices(seq_len: int) -> np.ndarray:
    rng = np.random.default_rng(0)
    pa = rng.uniform(0.0, 1.0, size=6)
    idx_list = []
    L = seq_len
    if pa[0] < _P:
        start = int(rng.uniform(0.0, L * 0.1))
        idx = np.arange(start, L, dtype=np.int64)
        idx_list.append(idx)
        L = idx.shape[0]
    if pa[1] < _P:
        end = int(rng.uniform(0.0, L * 0.1))
        idx = np.arange(0, L - end, dtype=np.int64)
        idx_list.append(idx)
        L = idx.shape[0]
    if pa[2] < _P:
        delta = float(np.float16(rng.uniform(0.8, 1.2)))
        idx = np.floor(np.arange(0.0, L, delta)).astype(np.int64)
        idx = np.clip(idx, 0, L - 1)
        idx_list.append(idx)
        L = idx.shape[0]
    if pa[3] < _P:
        margin = int(0.1 * L)
        center = int(rng.uniform(margin, L - margin))
        delta = float(np.float16(rng.uniform(0.5, 1.5)))
        mid = np.arange(center - margin, center + margin, delta).astype(np.int64)
        mid = np.clip(mid, 0, L - 1)
        idx = np.concatenate([
            np.arange(0, center - margin, dtype=np.int64),
            mid,
            np.arange(center + margin, L, dtype=np.int64),
        ])
        idx_list.append(idx)
        L = idx.shape[0]
    if pa[4] < _P:
        margin = int(0.1 * L)
        center = int(rng.uniform(margin, L - margin))
        mid = rng.permutation(np.arange(center - margin, center + margin, dtype=np.int64))
        idx = np.concatenate([
            np.arange(0, center - margin, dtype=np.int64),
            mid,
            np.arange(center + margin, L, dtype=np.int64),
        ])
        idx_list.append(idx)
        L = idx.shape[0]
    if pa[4] < _P:
        margin = int(0.1 * L)
        center = int(rng.uniform(margin, L - margin))
        idx = np.concatenate([
            np.arange(0, center - margin, dtype=np.int64),
            np.arange(center - margin, center + margin, dtype=np.int64)[::-1],
            np.arange(center + margin, L, dtype=np.int64),
        ])
        idx_list.append(idx)
        L = idx.shape[0]
    final = idx_list[0]
    for idx in idx_list[1:]:
        final = final[idx]
    return final.astype(np.int32)


_SEQ_LEN = 4096
_FEAT = 512
_IDX_NP = _plan_indices(_SEQ_LEN)
_OUT_LEN = int(_IDX_NP.shape[0])

_NUM_WORKERS = 32
_ROWS_PER_WORKER = -(-_OUT_LEN // (8 * _NUM_WORKERS)) * 8
assert _ROWS_PER_WORKER <= 128
_LAST_BASE = _OUT_LEN - _ROWS_PER_WORKER
assert _LAST_BASE % 8 == 0 and _OUT_LEN % 8 == 0

_CHUNK_OFFS = (0, _ROWS_PER_WORKER)
_N_CHUNKS = len(_CHUNK_OFFS) - 1
assert all(o % 8 == 0 for o in _CHUNK_OFFS)

_LANES = 16
_GEN_LEN = -(-_ROWS_PER_WORKER // _LANES) * _LANES


def _formula_params():
    rng = np.random.default_rng(0)
    pa = rng.uniform(0.0, 1.0, size=6)
    assert pa[0] >= _P and pa[1] < _P and pa[2] < _P and pa[3] < _P and pa[4] >= _P
    L = _SEQ_LEN
    end = int(rng.uniform(0.0, L * 0.1))
    L1 = L - end
    delta1 = float(np.float16(rng.uniform(0.8, 1.2)))
    L2 = len(np.arange(0.0, L1, delta1))
    margin = int(0.1 * L2)
    center = int(rng.uniform(margin, L2 - margin))
    delta3 = float(np.float16(rng.uniform(0.5, 1.5)))
    midlen = len(np.arange(center - margin, center + margin, delta3))
    p1 = center - margin
    shift = 2 * margin - midlen
    return dict(L1=L1, L2=L2, delta1=delta1, delta3=delta3,
                p1=p1, midlen=midlen, shift=shift)


_FP = _formula_params()


def _simulate_f32(j):
    f32 = np.float32
    jf = j.astype(f32)
    mid = (jf - f32(_FP["p1"])) * f32(_FP["delta3"]) + f32(_FP["p1"])
    v_mid = np.minimum(mid.astype(np.int32), _FP["L2"] - 1)
    v = np.where(j < _FP["p1"], j,
                 np.where(j < _FP["p1"] + _FP["midlen"], v_mid,
                          j + _FP["shift"]))
    return np.minimum((v.astype(f32) * f32(_FP["delta1"])).astype(np.int32),
                      _FP["L1"] - 1)


_ALL_J = np.arange(0, _OUT_LEN + (_GEN_LEN - _ROWS_PER_WORKER), dtype=np.int32)
_SIM = _simulate_f32(_ALL_J)
assert np.array_equal(_SIM[:_OUT_LEN], _IDX_NP)
assert _SIM.min() >= 0 and _SIM.max() < _SEQ_LEN


def _gen_indices(idx_v, base):
    f32, s32 = jnp.float32, jnp.int32
    for t in range(_GEN_LEN // _LANES):
        j = lax.iota(s32, _LANES) + (base + t * _LANES)
        jf = j.astype(f32)
        mid = (jf - f32(_FP["p1"])) * f32(_FP["delta3"]) + f32(_FP["p1"])
        v_mid = jnp.minimum(mid.astype(s32), _FP["L2"] - 1)
        v = jnp.where(j < _FP["p1"], j,
                      jnp.where(j < _FP["p1"] + _FP["midlen"], v_mid,
                                j + _FP["shift"]))
        fin = jnp.minimum((v.astype(f32) * f32(_FP["delta1"])).astype(s32),
                          _FP["L1"] - 1)
        idx_v[pl.ds(t * _LANES, _LANES)] = fin


def _sc_gather(a_hbm, b_hbm, out_hbm, idx_v, rows_a, rows_b, gsems, wsems):
    info = plsc.get_sparse_core_info()
    wid = lax.axis_index("s") * info.num_cores + lax.axis_index("c")
    base = jnp.where(wid == _NUM_WORKERS - 1, _LAST_BASE,
                     wid * _ROWS_PER_WORKER)
    _gen_indices(idx_v, base)
    gathers = []
    for c in range(_N_CHUNKS):
        off, n = _CHUNK_OFFS[c], _CHUNK_OFFS[c + 1] - _CHUNK_OFFS[c]
        sl = pl.ds(off, n)
        for j, (src, buf) in enumerate(((a_hbm, rows_a), (b_hbm, rows_b))):
            cp = pltpu.async_copy(src.at[idx_v.at[sl]], buf.at[sl],
                                  gsems.at[c * 2 + j])
            gathers.append((cp, buf, off, n, j))
    writes = []
    for k, (cp, buf, off, n, j) in enumerate(gathers):
        cp.wait()
        writes.append(pltpu.async_copy(buf.at[pl.ds(off, n)],
                                       out_hbm.at[pl.ds(base + off + j * _OUT_LEN, n)],
                                       wsems.at[k]))
    for w in writes:
        w.wait()


@jax.jit
def kernel(seq_a, seq_b):
    mesh = plsc.VectorSubcoreMesh(core_axis_name="c", subcore_axis_name="s")
    out = pl.kernel(
        _sc_gather,
        mesh=mesh,
        out_type=jax.ShapeDtypeStruct((2 * _OUT_LEN, _FEAT), jnp.float32),
        scratch_types=[
            pltpu.VMEM((_GEN_LEN,), jnp.int32),
            pltpu.VMEM((_ROWS_PER_WORKER, _FEAT), jnp.float32),
            pltpu.VMEM((_ROWS_PER_WORKER, _FEAT), jnp.float32),
            pltpu.SemaphoreType.DMA((_N_CHUNKS * 2,)),
            pltpu.SemaphoreType.DMA((_N_CHUNKS * 2,)),
        ],
    )(seq_a, seq_b)
    return out.reshape(2, _OUT_LEN, _FEAT)
